# baseline (speedup 1.0000x reference)
import functools

import jax
import jax.numpy as jnp
from jax import lax
from jax.experimental import pallas as pl
from jax.experimental.pallas import tpu as pltpu
from jax.experimental.pallas import tpu_sc as plsc

N_NODES = 10000
N_PAD = 10240
N_EDGES = 320000
CHUNK = 128
N_CORES = 2
N_SUBCORES = 16
N_WORKERS = N_CORES * N_SUBCORES
E_PAD = 327680
CHUNKS_PER_W = E_PAD // (N_WORKERS * CHUNK)
ROWS_PER_TILE = N_PAD // N_SUBCORES
DEG_W = 128

_MESH = plsc.VectorSubcoreMesh(core_axis_name="c", subcore_axis_name="s")


DEG_DEPTH = 8


def _sc_deg_kernel():
  @functools.partial(
      pl.kernel,
      mesh=_MESH,
      out_type=jax.ShapeDtypeStruct((N_CORES, N_PAD, DEG_W), jnp.float32),
      scratch_types=[
          pltpu.VMEM((CHUNKS_PER_W, CHUNK), jnp.int32),
          pltpu.VMEM((CHUNK, DEG_W), jnp.float32),
          pltpu.VMEM_SHARED((N_PAD, DEG_W), jnp.float32),
      ]
      + [pltpu.SemaphoreType.DMA] * DEG_DEPTH,
  )
  def k(dst_hbm, ones_hbm, zeros_hbm, out_hbm, idxd_all, ones_v, deg_sh, *sems):
    c = lax.axis_index("c")
    s = lax.axis_index("s")
    wid = c * N_SUBCORES + s
    pltpu.sync_copy(dst_hbm.at[pl.ds(wid * CHUNKS_PER_W, CHUNKS_PER_W)], idxd_all)
    pltpu.sync_copy(zeros_hbm, deg_sh.at[pl.ds(s * ROWS_PER_TILE, ROWS_PER_TILE)])
    pltpu.sync_copy(ones_hbm, ones_v)
    plsc.subcore_barrier()

    def scat(j, b):
      return pltpu.make_async_copy(ones_v, deg_sh.at[idxd_all.at[j]], sems[b])

    def grp(g, carry):
      for b in range(DEG_DEPTH):
        j = g * DEG_DEPTH + b

        @pl.when(g > 0)
        def _():
          scat(j, b).wait()

        scat(j, b).start(add=True)
      return carry

    n_groups = CHUNKS_PER_W // DEG_DEPTH
    lax.fori_loop(0, n_groups, grp, 0)
    for b in range(DEG_DEPTH):
      scat((n_groups - 1) * DEG_DEPTH + b, b).wait()
    plsc.subcore_barrier()
    sl = pl.ds(s * ROWS_PER_TILE, ROWS_PER_TILE)
    pltpu.sync_copy(deg_sh.at[sl], out_hbm.at[c, sl])

  return k


SCHUNK = 64
NBUF = 4
TOTAL_CHUNKS = E_PAD // SCHUNK
Q0 = 320
Q1 = (TOTAL_CHUNKS - N_SUBCORES * Q0) // N_SUBCORES
STAGE_CHUNKS = 16
N_GROUPS = STAGE_CHUNKS // NBUF


def _sc_scatter_kernel(d_model):
  @functools.partial(
      pl.kernel,
      mesh=_MESH,
      out_type=jax.ShapeDtypeStruct((N_CORES, N_PAD, d_model), jnp.float32),
      scratch_types=[
          pltpu.VMEM((STAGE_CHUNKS, SCHUNK), jnp.int32),
          pltpu.VMEM((STAGE_CHUNKS, SCHUNK), jnp.int32),
          pltpu.VMEM_SHARED((N_PAD, d_model), jnp.float32),
      ]
      + [pltpu.VMEM((SCHUNK, d_model), jnp.float32)] * NBUF
      + [pltpu.SemaphoreType.DMA] * (2 * NBUF),
  )
  def k(hs_hbm, src_hbm, dst_hbm, zeros_hbm, out_hbm,
        idxs_half, idxd_half, acc_sh, *bufs_and_sems):
    rows = bufs_and_sems[:NBUF]
    gsem = bufs_and_sems[NBUF:2 * NBUF]
    ssem = bufs_and_sems[2 * NBUF:]
    c = lax.axis_index("c")
    s = lax.axis_index("s")

    pltpu.sync_copy(zeros_hbm, rows[0])
    for i in range(ROWS_PER_TILE // SCHUNK):
      pltpu.sync_copy(rows[0], acc_sh.at[pl.ds(s * ROWS_PER_TILE + i * SCHUNK, SCHUNK)])
    plsc.subcore_barrier()

    def gather(j, b):
      return pltpu.make_async_copy(hs_hbm.at[idxs_half.at[j]], rows[b], gsem[b])

    def scatter(j, b):
      return pltpu.make_async_copy(rows[b], acc_sh.at[idxd_half.at[j]], ssem[b])

    def pipeline(worker_base):
      def stage(hh, carry):
        base_row = worker_base + hh * STAGE_CHUNKS
        pltpu.sync_copy(src_hbm.at[pl.ds(base_row, STAGE_CHUNKS)], idxs_half)
        pltpu.sync_copy(dst_hbm.at[pl.ds(base_row, STAGE_CHUNKS)], idxd_half)
        for b in range(NBUF):
          gather(b, b).start()

        def grp(g, carry2):
          base = g * NBUF
          for b in range(NBUF):
            gather(base + b, b).wait()
            scatter(base + b, b).start(add=True)

          @pl.when(g < N_GROUPS - 1)
          def _():
            for b in range(NBUF):
              scatter(base + b, b).wait()
              gather(base + NBUF + b, b).start()

          return carry2

        lax.fori_loop(0, N_GROUPS, grp, 0)
        for b in range(NBUF):
          scatter((N_GROUPS - 1) * NBUF + b, b).wait()
        return carry

      return stage

    @pl.when(c == 0)
    def _():
      lax.fori_loop(0, Q0 // STAGE_CHUNKS, pipeline(s * Q0), 0)

    @pl.when(c == 1)
    def _():
      lax.fori_loop(0, Q1 // STAGE_CHUNKS, pipeline(N_SUBCORES * Q0 + s * Q1), 0)

    plsc.subcore_barrier()
    sl = pl.ds(s * ROWS_PER_TILE, ROWS_PER_TILE)
    pltpu.sync_copy(acc_sh.at[sl], out_hbm.at[c, sl])

  return k


def _tc_first(deg_parts, x, w):
  def body(degp_ref, x_ref, w_ref, dis_ref, h_ref, hs_ref):
    degp = degp_ref[...]
    deg = degp.sum(0)[:, 0:1] + 2.0
    dis = lax.rsqrt(deg)
    h = jnp.dot(x_ref[...], w_ref[...], preferred_element_type=jnp.float32)
    dis_ref[...] = dis
    h_ref[...] = h
    hs_ref[...] = dis * h

  return pl.pallas_call(
      body,
      out_shape=[
          jax.ShapeDtypeStruct((N_PAD, 1), jnp.float32),
          jax.ShapeDtypeStruct((N_PAD, w.shape[1]), jnp.float32),
          jax.ShapeDtypeStruct((N_PAD, w.shape[1]), jnp.float32),
      ],
  )(deg_parts, x, w)


def _tc_mid(acc_parts, h_prev, dis, b, w, hs_width=None):
  d = w.shape[1]
  hs_width = hs_width or d

  def body(accp_ref, hp_ref, dis_ref, b_ref, w_ref, h_ref, hs_ref):
    accp = accp_ref[...]
    dis = dis_ref[...]
    pre = dis * accp.sum(0) + (2.0 * dis * dis) * hp_ref[...] + b_ref[...]
    y = jnp.maximum(pre, 0.0)
    h = jnp.dot(y, w_ref[...], preferred_element_type=jnp.float32)
    h_ref[...] = h
    hs = dis * h
    if hs_width != d:
      hs = jnp.concatenate(
          [hs, jnp.zeros((N_PAD, hs_width - d), jnp.float32)], axis=1)
    hs_ref[...] = hs

  return pl.pallas_call(
      body,
      out_shape=[
          jax.ShapeDtypeStruct((N_PAD, d), jnp.float32),
          jax.ShapeDtypeStruct((N_PAD, hs_width), jnp.float32),
      ],
  )(acc_parts, h_prev, dis, b, w)


def _tc_last(acc_parts, h_prev, dis, b):
  d = b.shape[1]

  def body(accp_ref, hp_ref, dis_ref, b_ref, out_ref):
    accp = accp_ref[...]
    dis = dis_ref[...]
    acc = accp.sum(0)[:, :d]
    pre = dis * acc + (2.0 * dis * dis) * hp_ref[...] + b_ref[...]
    out_ref[...] = jnp.tanh(pre)

  return pl.pallas_call(
      body,
      out_shape=jax.ShapeDtypeStruct((N_PAD, d), jnp.float32),
  )(acc_parts, h_prev, dis, b)


def kernel(x, edge_index, W_in, b_in, W_h0, b_h0, W_h1, b_h1, W_out, b_out):
  d_hid = W_in.shape[1]
  d_out = W_out.shape[1]

  ei = edge_index.astype(jnp.int32)
  n_pad_e = E_PAD - ei.shape[1]
  pad_src = jnp.full((n_pad_e,), N_NODES, dtype=jnp.int32)
  pad_dst = N_NODES + (jnp.arange(n_pad_e, dtype=jnp.int32) % (N_PAD - N_NODES))
  src_flat = jnp.concatenate([ei[0], pad_src])
  dst_flat = jnp.concatenate([ei[1], pad_dst])
  src2d = src_flat.reshape(E_PAD // CHUNK, CHUNK)
  dst2d = dst_flat.reshape(E_PAD // CHUNK, CHUNK)
  src2s = src_flat.reshape(E_PAD // SCHUNK, SCHUNK)
  dst2s = dst_flat.reshape(E_PAD // SCHUNK, SCHUNK)

  x_pad = jnp.zeros((N_PAD, x.shape[1]), jnp.float32).at[:N_NODES].set(x)
  zeros_deg = jnp.zeros((ROWS_PER_TILE, DEG_W), jnp.float32)
  ones_deg = jnp.ones((CHUNK, DEG_W), jnp.float32)
  zeros_hid = jnp.zeros((SCHUNK, d_hid), jnp.float32)

  deg_parts = _sc_deg_kernel()(dst2d, ones_deg, zeros_deg)

  dis, h, hs = _tc_first(deg_parts, x_pad, W_in)
  scat_hid = _sc_scatter_kernel(d_hid)
  acc = scat_hid(hs, src2s, dst2s, zeros_hid)

  h2, hs2 = _tc_mid(acc, h, dis, b_in.reshape(1, -1), W_h0)
  acc2 = scat_hid(hs2, src2s, dst2s, zeros_hid)
  h3, hs3 = _tc_mid(acc2, h2, dis, b_h0.reshape(1, -1), W_h1)
  acc3 = scat_hid(hs3, src2s, dst2s, zeros_hid)

  h4, hs4 = _tc_mid(acc3, h3, dis, b_h1.reshape(1, -1), W_out, hs_width=d_hid)
  acc4 = scat_hid(hs4, src2s, dst2s, zeros_hid)
  out = _tc_last(acc4, h4, dis, b_out.reshape(1, -1))

  return out[:N_NODES]


# speedup vs baseline: 1.3146x; 1.3146x over previous
import functools

import jax
import jax.numpy as jnp
from jax import lax
from jax.experimental import pallas as pl
from jax.experimental.pallas import tpu as pltpu
from jax.experimental.pallas import tpu_sc as plsc

N_NODES = 10000
N_PAD = 10240
N_EDGES = 320000
CHUNK = 128
N_CORES = 2
N_SUBCORES = 16
N_WORKERS = N_CORES * N_SUBCORES
E_PAD = 327680
CHUNKS_PER_W = E_PAD // (N_WORKERS * CHUNK)
ROWS_PER_TILE = N_PAD // N_SUBCORES
DEG_W = 128

_MESH = plsc.VectorSubcoreMesh(core_axis_name="c", subcore_axis_name="s")


DEG_DEPTH = 8


def _sc_deg_kernel():
  @functools.partial(
      pl.kernel,
      mesh=_MESH,
      out_type=jax.ShapeDtypeStruct((N_CORES, N_PAD, DEG_W), jnp.float32),
      scratch_types=[
          pltpu.VMEM((CHUNKS_PER_W, CHUNK), jnp.int32),
          pltpu.VMEM((CHUNK, DEG_W), jnp.float32),
          pltpu.VMEM_SHARED((N_PAD, DEG_W), jnp.float32),
      ]
      + [pltpu.SemaphoreType.DMA] * DEG_DEPTH,
  )
  def k(dst_hbm, ones_hbm, zeros_hbm, out_hbm, idxd_all, ones_v, deg_sh, *sems):
    c = lax.axis_index("c")
    s = lax.axis_index("s")
    wid = c * N_SUBCORES + s
    pltpu.sync_copy(dst_hbm.at[pl.ds(wid * CHUNKS_PER_W, CHUNKS_PER_W)], idxd_all)
    pltpu.sync_copy(zeros_hbm, deg_sh.at[pl.ds(s * ROWS_PER_TILE, ROWS_PER_TILE)])
    pltpu.sync_copy(ones_hbm, ones_v)
    plsc.subcore_barrier()

    def scat(j, b):
      return pltpu.make_async_copy(ones_v, deg_sh.at[idxd_all.at[j]], sems[b])

    def grp(g, carry):
      for b in range(DEG_DEPTH):
        j = g * DEG_DEPTH + b

        @pl.when(g > 0)
        def _():
          scat(j, b).wait()

        scat(j, b).start(add=True)
      return carry

    n_groups = CHUNKS_PER_W // DEG_DEPTH
    lax.fori_loop(0, n_groups, grp, 0)
    for b in range(DEG_DEPTH):
      scat((n_groups - 1) * DEG_DEPTH + b, b).wait()
    plsc.subcore_barrier()
    sl = pl.ds(s * ROWS_PER_TILE, ROWS_PER_TILE)
    pltpu.sync_copy(deg_sh.at[sl], out_hbm.at[c, sl])

  return k


SCHUNK = 64
NBUF = 4
TOTAL_CHUNKS = E_PAD // SCHUNK
Q0 = 304
Q1 = (TOTAL_CHUNKS - N_SUBCORES * Q0) // N_SUBCORES
STAGE_CHUNKS = 16
N_GROUPS = STAGE_CHUNKS // NBUF


def _sc_scatter_kernel(d_model):
  @functools.partial(
      pl.kernel,
      mesh=_MESH,
      out_type=jax.ShapeDtypeStruct((N_CORES, N_PAD, d_model), jnp.float32),
      scratch_types=[
          pltpu.VMEM((STAGE_CHUNKS, SCHUNK), jnp.int32),
          pltpu.VMEM((STAGE_CHUNKS, SCHUNK), jnp.int32),
          pltpu.VMEM_SHARED((N_PAD, d_model), jnp.float32),
      ]
      + [pltpu.VMEM((SCHUNK, d_model), jnp.float32)] * NBUF
      + [pltpu.SemaphoreType.DMA] * (2 * NBUF),
  )
  def k(hs_hbm, src_hbm, dst_hbm, zeros_hbm, out_hbm,
        idxs_half, idxd_half, acc_sh, *bufs_and_sems):
    rows = bufs_and_sems[:NBUF]
    gsem = bufs_and_sems[NBUF:2 * NBUF]
    ssem = bufs_and_sems[2 * NBUF:]
    c = lax.axis_index("c")
    s = lax.axis_index("s")

    pltpu.sync_copy(zeros_hbm, rows[0])
    for i in range(ROWS_PER_TILE // SCHUNK):
      pltpu.sync_copy(rows[0], acc_sh.at[pl.ds(s * ROWS_PER_TILE + i * SCHUNK, SCHUNK)])
    plsc.subcore_barrier()

    def gather(j, b):
      return pltpu.make_async_copy(hs_hbm.at[idxs_half.at[j]], rows[b], gsem[b])

    def scatter(j, b):
      return pltpu.make_async_copy(rows[b], acc_sh.at[idxd_half.at[j]], ssem[b])

    def pipeline(worker_base):
      def stage(hh, carry):
        base_row = worker_base + hh * STAGE_CHUNKS
        pltpu.sync_copy(src_hbm.at[pl.ds(base_row, STAGE_CHUNKS)], idxs_half)
        pltpu.sync_copy(dst_hbm.at[pl.ds(base_row, STAGE_CHUNKS)], idxd_half)
        for b in range(NBUF):
          gather(b, b).start()

        def grp(g, carry2):
          base = g * NBUF
          for b in range(NBUF):
            gather(base + b, b).wait()
            scatter(base + b, b).start(add=True)

          @pl.when(g < N_GROUPS - 1)
          def _():
            for b in range(NBUF):
              scatter(base + b, b).wait()
              gather(base + NBUF + b, b).start()

          return carry2

        lax.fori_loop(0, N_GROUPS, grp, 0)
        for b in range(NBUF):
          scatter((N_GROUPS - 1) * NBUF + b, b).wait()
        return carry

      return stage

    @pl.when(c == 0)
    def _():
      lax.fori_loop(0, Q0 // STAGE_CHUNKS, pipeline(s * Q0), 0)

    @pl.when(c == 1)
    def _():
      lax.fori_loop(0, Q1 // STAGE_CHUNKS, pipeline(N_SUBCORES * Q0 + s * Q1), 0)

    plsc.subcore_barrier()
    sl = pl.ds(s * ROWS_PER_TILE, ROWS_PER_TILE)
    pltpu.sync_copy(acc_sh.at[sl], out_hbm.at[c, sl])

  return k


def _tc_first(deg_parts, x, w):
  def body(degp_ref, x_ref, w_ref, dis_ref, h_ref, hs_ref):
    degp = degp_ref[...]
    deg = degp.sum(0)[:, 0:1] + 2.0
    dis = lax.rsqrt(deg)
    h = jnp.dot(x_ref[...], w_ref[...], preferred_element_type=jnp.float32)
    dis_ref[...] = dis
    h_ref[...] = h
    hs_ref[...] = dis * h

  return pl.pallas_call(
      body,
      out_shape=[
          jax.ShapeDtypeStruct((N_PAD, 1), jnp.float32),
          jax.ShapeDtypeStruct((N_PAD, w.shape[1]), jnp.float32),
          jax.ShapeDtypeStruct((N_PAD, w.shape[1]), jnp.float32),
      ],
  )(deg_parts, x, w)


def _tc_mid(acc_parts, h_prev, dis, b, w, hs_width=None):
  d = w.shape[1]
  hs_width = hs_width or d

  def body(accp_ref, hp_ref, dis_ref, b_ref, w_ref, h_ref, hs_ref):
    accp = accp_ref[...]
    dis = dis_ref[...]
    pre = dis * accp.sum(0) + (2.0 * dis * dis) * hp_ref[...] + b_ref[...]
    y = jnp.maximum(pre, 0.0)
    h = jnp.dot(y, w_ref[...], preferred_element_type=jnp.float32)
    h_ref[...] = h
    hs = dis * h
    if hs_width != d:
      hs = jnp.concatenate(
          [hs, jnp.zeros((N_PAD, hs_width - d), jnp.float32)], axis=1)
    hs_ref[...] = hs

  return pl.pallas_call(
      body,
      out_shape=[
          jax.ShapeDtypeStruct((N_PAD, d), jnp.float32),
          jax.ShapeDtypeStruct((N_PAD, hs_width), jnp.float32),
      ],
  )(acc_parts, h_prev, dis, b, w)


def _tc_last(acc_parts, h_prev, dis, b):
  d = b.shape[1]

  def body(accp_ref, hp_ref, dis_ref, b_ref, out_ref):
    accp = accp_ref[...]
    dis = dis_ref[...]
    acc = accp.sum(0)[:, :d]
    pre = dis * acc + (2.0 * dis * dis) * hp_ref[...] + b_ref[...]
    out_ref[...] = jnp.tanh(pre)

  return pl.pallas_call(
      body,
      out_shape=jax.ShapeDtypeStruct((N_PAD, d), jnp.float32),
  )(acc_parts, h_prev, dis, b)


def kernel(x, edge_index, W_in, b_in, W_h0, b_h0, W_h1, b_h1, W_out, b_out):
  d_hid = W_in.shape[1]
  d_out = W_out.shape[1]

  ei = edge_index.astype(jnp.int32)
  n_pad_e = E_PAD - ei.shape[1]
  pad_src = jnp.full((n_pad_e,), N_NODES, dtype=jnp.int32)
  pad_dst = N_NODES + (jnp.arange(n_pad_e, dtype=jnp.int32) % (N_PAD - N_NODES))
  src_flat = jnp.concatenate([ei[0], pad_src])
  dst_flat = jnp.concatenate([ei[1], pad_dst])
  src2d = src_flat.reshape(E_PAD // CHUNK, CHUNK)
  dst2d = dst_flat.reshape(E_PAD // CHUNK, CHUNK)
  src2s = src_flat.reshape(E_PAD // SCHUNK, SCHUNK)
  dst2s = dst_flat.reshape(E_PAD // SCHUNK, SCHUNK)

  x_pad = jnp.zeros((N_PAD, x.shape[1]), jnp.float32).at[:N_NODES].set(x)
  zeros_deg = jnp.zeros((ROWS_PER_TILE, DEG_W), jnp.float32)
  ones_deg = jnp.ones((CHUNK, DEG_W), jnp.float32)
  zeros_hid = jnp.zeros((SCHUNK, d_hid), jnp.float32)

  deg_parts = _sc_deg_kernel()(dst2d, ones_deg, zeros_deg)

  dis, h, hs = _tc_first(deg_parts, x_pad, W_in)
  scat_hid = _sc_scatter_kernel(d_hid)
  acc = scat_hid(hs, src2s, dst2s, zeros_hid)

  h2, hs2 = _tc_mid(acc, h, dis, b_in.reshape(1, -1), W_h0)
  acc2 = scat_hid(hs2, src2s, dst2s, zeros_hid)
  h3, hs3 = _tc_mid(acc2, h2, dis, b_h0.reshape(1, -1), W_h1)
  acc3 = scat_hid(hs3, src2s, dst2s, zeros_hid)

  h4, hs4 = _tc_mid(acc3, h3, dis, b_h1.reshape(1, -1), W_out, hs_width=d_hid)
  acc4 = scat_hid(hs4, src2s, dst2s, zeros_hid)
  out = _tc_last(acc4, h4, dis, b_out.reshape(1, -1))

  return out[:N_NODES]
